# baseline (speedup 1.0000x reference)
import functools

import jax
import jax.numpy as jnp
from jax import lax
from jax.experimental import pallas as pl
from jax.experimental.pallas import tpu as pltpu
from jax.experimental.pallas import tpu_sc as plsc

MAXLEN = 200
EMBED_DIM = 64
BATCH = 1024
VOCAB = 1000000

NUM_WORKERS = 32
BBLK = 128
NBB = BATCH // BBLK
UNITS = MAXLEN * NBB
U_PER_W = UNITS // NUM_WORKERS
NCHUNK = BBLK // 16


def _sc_body(idxT_hbm, tok2_hbm, pos_hbm, out_hbm,
             idx_raw, idx_g, base_v, pairs_v, slab_v, pos_v, sem):
    wid = lax.axis_index("s") * 2 + lax.axis_index("c")
    u0 = wid * U_PER_W

    pltpu.sync_copy(pos_hbm, pos_v)

    def do_unit(i, carry):
        u = u0 + i
        p = u // NBB
        bb = u % NBB

        pltpu.sync_copy(idxT_hbm.at[p, pl.ds(bb * BBLK, BBLK)], idx_raw)
        for k in range(NCHUNK):
            sl = pl.ds(k * 16, 16)
            v16 = idx_raw[sl]
            idx_g[sl] = lax.shift_right_logical(v16, 1)
            base_v[sl] = lax.shift_left(jnp.bitwise_and(v16, 1), 6)
        pltpu.async_copy(tok2_hbm.at[idx_g], pairs_v, sem).wait()

        def k_iter(k, carry2):
            sl = pl.ds(k * 16, 16)
            half16 = base_v[sl]
            row16 = lax.iota(jnp.int32, 16) + (k * 16)
            for cc in range(EMBED_DIM // 16):
                pos16 = pos_v[p, pl.ds(cc * 16, 16)]
                for cl in range(16):
                    c = cc * 16 + cl
                    val = plsc.load_gather(pairs_v, [row16, half16 + c])
                    slab_v[c, sl] = val + pos16[cl]
            return carry2

        lax.fori_loop(0, NCHUNK, k_iter, 0)

        pltpu.sync_copy(slab_v, out_hbm.at[p, :, pl.ds(bb * BBLK, BBLK)])
        return carry

    lax.fori_loop(0, U_PER_W, do_unit, 0)


def kernel(inputs, token_table, pos_table):
    idxT = inputs.T.astype(jnp.int32)
    tok2 = token_table.reshape(VOCAB // 2, 2 * EMBED_DIM)
    mesh = plsc.VectorSubcoreMesh(core_axis_name="c", subcore_axis_name="s")
    k = functools.partial(
        pl.kernel,
        out_type=jax.ShapeDtypeStruct((MAXLEN, EMBED_DIM, BATCH), jnp.float32),
        mesh=mesh,
        scratch_types=[
            pltpu.VMEM((BBLK,), jnp.int32),
            pltpu.VMEM((BBLK,), jnp.int32),
            pltpu.VMEM((BBLK,), jnp.int32),
            pltpu.VMEM((BBLK, 128), jnp.float32),
            pltpu.VMEM((EMBED_DIM, BBLK), jnp.float32),
            pltpu.VMEM((MAXLEN, EMBED_DIM), jnp.float32),
            pltpu.SemaphoreType.DMA,
        ],
        compiler_params=pltpu.CompilerParams(
            use_tc_tiling_on_sc=True, needs_layout_passes=False
        ),
    )(_sc_body)
    out = k(idxT, tok2, pos_table)
    return jnp.transpose(out, (2, 0, 1))
